# baseline (speedup 1.0000x reference)
import jax
import jax.numpy as jnp
from jax.experimental import pallas as pl

H = 128
N_IND = 50000
N_FAC = 5000
LAYERS = 3
DG = 8
DP = 12


def _sage(x_src, x_dst, ei, Wl, bl, Wr, n_dst):
    msg = jnp.take(x_src, ei[0], axis=0)
    agg = jax.ops.segment_sum(msg, ei[1], num_segments=n_dst)
    cnt = jax.ops.segment_sum(jnp.ones((ei.shape[1],), jnp.float32), ei[1], num_segments=n_dst)
    mean = agg / jnp.clip(cnt, 1.0)[:, None]
    return mean @ Wl + bl + x_dst @ Wr


def _ln(x, g, b):
    m = x.mean(-1, keepdims=True)
    v = ((x - m) ** 2).mean(-1, keepdims=True)
    return (x - m) / jnp.sqrt(v + 1e-5) * g + b


def _heads_body(hf_ref, gpp_ref, pooled_ref,
                aw0_ref, ab0_ref, aw1_ref, ab1_ref, aw2_ref, ab2_ref,
                cw0_ref, cb0_ref, cw1_ref, cb1_ref, cw2_ref, cb2_ref,
                logits_ref, value_ref):
    hf = hf_ref[...]
    gpp = gpp_ref[...]
    a = hf @ aw0_ref[:H, :] + gpp @ aw0_ref[H:, :] + ab0_ref[...]
    a = jnp.maximum(a, 0.0)
    a = jnp.maximum(a @ aw1_ref[...] + ab1_ref[...], 0.0)
    logits_ref[...] = a @ aw2_ref[...] + ab2_ref[...]
    cc = pooled_ref[...]
    c = cc @ cw0_ref[:2 * H, :] + gpp @ cw0_ref[2 * H:, :] + cb0_ref[...]
    c = jnp.maximum(c, 0.0)
    c = jnp.maximum(c @ cw1_ref[...] + cb1_ref[...], 0.0)
    value_ref[...] = c @ cw2_ref[...] + cb2_ref[...]


def kernel(x_individual, x_facility, edge_index_interacts, edge_index_visits, edge_index_visited_by, edge_index_connects, global_features, problem_params, params):
    p = params
    h_i = x_individual @ p['proj_i_W'] + p['proj_i_b']
    h_f = x_facility @ p['proj_f_W'] + p['proj_f_b']
    for l in range(LAYERS):
        new_i = _sage(h_i, h_i, edge_index_interacts, p['W_l_%d_ii' % l], p['b_l_%d_ii' % l], p['W_r_%d_ii' % l], N_IND) \
              + _sage(h_f, h_i, edge_index_visited_by, p['W_l_%d_fi' % l], p['b_l_%d_fi' % l], p['W_r_%d_fi' % l], N_IND)
        new_f = _sage(h_i, h_f, edge_index_visits, p['W_l_%d_if' % l], p['b_l_%d_if' % l], p['W_r_%d_if' % l], N_FAC) \
              + _sage(h_f, h_f, edge_index_connects, p['W_l_%d_ff' % l], p['b_l_%d_ff' % l], p['W_r_%d_ff' % l], N_FAC)
        new_i = _ln(new_i, p['ln_i_g_%d' % l], p['ln_i_b_%d' % l])
        new_f = _ln(new_f, p['ln_f_g_%d' % l], p['ln_f_b_%d' % l])
        h_i = jax.nn.relu(h_i + new_i)
        h_f = jax.nn.relu(h_f + new_f)

    gpp = jnp.concatenate([global_features, problem_params])[None, :]
    pooled = jnp.concatenate([h_i.mean(axis=0), h_f.mean(axis=0)])[None, :]
    logits2, value2 = pl.pallas_call(
        _heads_body,
        out_shape=(jax.ShapeDtypeStruct((N_FAC, 1), jnp.float32),
                   jax.ShapeDtypeStruct((1, 1), jnp.float32)),
    )(h_f, gpp, pooled,
      p['actor_W0'], p['actor_b0'][None, :], p['actor_W1'], p['actor_b1'][None, :],
      p['actor_W2'], p['actor_b2'][None, :],
      p['critic_W0'], p['critic_b0'][None, :], p['critic_W1'], p['critic_b1'][None, :],
      p['critic_W2'], p['critic_b2'][None, :])
    return (logits2[:, 0], value2[0, 0])


# speedup vs baseline: 2.4288x; 2.4288x over previous
import functools

import jax
import jax.numpy as jnp
from jax import lax
from jax.experimental import pallas as pl
from jax.experimental.pallas import tpu as pltpu
from jax.experimental.pallas import tpu_sc as plsc

H = 128
CB = 32
N_IND = 50000
N_FAC = 5000
NIP = 51200
NFP = 5120
LAYERS = 3
DG = 8
DP = 12

NS = 16
EB = 2048
GB = 128

ACC_B = 5136
ACC_B_CNT = 5376

E_II = 802816
E_FI = 401408
E_IF = 401408
E_FF = 40960

_mesh = plsc.VectorSubcoreMesh(core_axis_name="c", subcore_axis_name="s")


def _pad_edges(ei, e_pad, n_src, trash_base, trash_spread):
    e = ei.shape[1]
    ar = jnp.arange(e_pad - e, dtype=jnp.int32)
    pad = jnp.stack([ar % n_src, trash_base + (ar % trash_spread)])
    return jnp.concatenate([ei, pad], axis=1)


def _fill_zeros_2d(buf):
    z = jnp.zeros((16,), jnp.float32)

    def row(r, _):
        for t in range(CB // 16):
            buf[r, pl.ds(t * 16, 16)] = z
        return 0
    lax.fori_loop(0, 128, row, 0)


def _fill_const_1d(buf, n, val):
    v = jnp.full((16,), val, jnp.float32)

    def body(r, _):
        buf[pl.ds(r * 16, 16)] = v
        return 0
    lax.fori_loop(0, n // 16, body, 0)


def _copy128(dst128, src_ref, off):
    for t in range(8):
        dst128[pl.ds(t * 16, 16)] = src_ref[pl.ds(off + t * 16, 16)]


def _strided_blocks(sid, n_blk):
    q, r = n_blk // NS, n_blk % NS
    return jnp.where(sid < r, q + 1, q)


def _agg_body(hic0, hic1, hic2, hic3, hfc0, hfc1, hfc2, hfc3,
              s_ii, d_ii, s_fi, d_fi, s_if, d_if, s_ff, d_ff,
              agg_ii, agg_fi, agg_if, agg_ff,
              acc, srcblk, dstblk, csrc128, cdst128, rowbuf, zbuf):
    cid = lax.axis_index("c")
    sid = lax.axis_index("s")
    _fill_zeros_2d(zbuf)

    def type_pass(table, src_ref, dst_ref, n_blk, out3d, kb, zrows, orows):
        zt = zrows // NS
        zbase = sid * zt

        def zero(r, _):
            pltpu.sync_copy(zbuf, acc.at[pl.ds(zbase + r * 128, 128)])
            return 0
        lax.fori_loop(0, zt // 128, zero, 0)
        if zt % 128:
            pltpu.sync_copy(zbuf.at[pl.ds(0, zt % 128)],
                            acc.at[pl.ds(zbase + (zt // 128) * 128, zt % 128)])
        plsc.subcore_barrier()

        my_n = _strided_blocks(sid, n_blk)

        def blk(k, _):
            base = (sid + k * NS) * EB
            pltpu.sync_copy(src_ref.at[pl.ds(base, EB)], srcblk)
            pltpu.sync_copy(dst_ref.at[pl.ds(base, EB)], dstblk)

            def win(t, _):
                _copy128(csrc128, srcblk, t * GB)
                _copy128(cdst128, dstblk, t * GB)
                pltpu.sync_copy(table.at[csrc128], rowbuf)
                pltpu.sync_copy(rowbuf, acc.at[cdst128], add=True)
                return 0
            lax.fori_loop(0, EB // GB, win, 0)
            return 0
        lax.fori_loop(0, my_n, blk, 0)
        plsc.subcore_barrier()
        rows = orows // NS
        pltpu.sync_copy(acc.at[pl.ds(sid * rows, rows)],
                        out3d.at[kb, pl.ds(sid * rows, rows)])
        plsc.subcore_barrier()

    @pl.when(cid == 0)
    def _():
        type_pass(hic0, s_ii, d_ii, E_II // EB, agg_ii, 0, NIP, NIP)
        type_pass(hic1, s_ii, d_ii, E_II // EB, agg_ii, 1, NIP, NIP)
        for kb, tbl in enumerate((hfc0, hfc1, hfc2, hfc3)):
            type_pass(tbl, s_fi, d_fi, E_FI // EB, agg_fi, kb, ACC_B, NFP)

    @pl.when(cid == 1)
    def _():
        type_pass(hic2, s_ii, d_ii, E_II // EB, agg_ii, 2, NIP, NIP)
        type_pass(hic3, s_ii, d_ii, E_II // EB, agg_ii, 3, NIP, NIP)
        for kb, tbl in enumerate((hic0, hic1, hic2, hic3)):
            type_pass(tbl, s_if, d_if, E_IF // EB, agg_if, kb, ACC_B, NFP)
        for kb, tbl in enumerate((hfc0, hfc1, hfc2, hfc3)):
            type_pass(tbl, s_ff, d_ff, E_FF // EB, agg_ff, kb, ACC_B, NFP)


_agg = functools.partial(
    pl.kernel, _agg_body, mesh=_mesh, name="sc_agg",
    compiler_params=pltpu.CompilerParams(use_tc_tiling_on_sc=False),
    out_type=(jax.ShapeDtypeStruct((4, NIP, CB), jnp.float32),
              jax.ShapeDtypeStruct((4, NFP, CB), jnp.float32),
              jax.ShapeDtypeStruct((4, NFP, CB), jnp.float32),
              jax.ShapeDtypeStruct((4, NFP, CB), jnp.float32)),
    scratch_types=[
        pltpu.VMEM_SHARED((NIP, CB), jnp.float32),
        pltpu.VMEM((EB,), jnp.int32),
        pltpu.VMEM((EB,), jnp.int32),
        pltpu.VMEM((GB,), jnp.int32),
        pltpu.VMEM((GB,), jnp.int32),
        pltpu.VMEM((GB, CB), jnp.float32),
        pltpu.VMEM((128, CB), jnp.float32),
    ])()


def _cnt_body(dst_ii, dst_fi, dst_if, dst_ff,
              cnt_ii, cnt_fi, cnt_if, cnt_ff,
              acc1d, dstblk, cdst128, ones128, zbuf1d, cntbuf):
    cid = lax.axis_index("c")
    sid = lax.axis_index("s")
    _fill_const_1d(ones128, GB, 1.0)
    _fill_const_1d(zbuf1d, 3296, 0.0)

    def count_type(dst_ref, n_blk, out, acc_rows, out_rows):
        ztile = acc_rows // NS
        pltpu.sync_copy(zbuf1d.at[pl.ds(0, ztile)],
                        acc1d.at[pl.ds(sid * ztile, ztile)])
        plsc.subcore_barrier()
        my_n = _strided_blocks(sid, n_blk)

        def blk(k, _):
            base = (sid + k * NS) * EB
            pltpu.sync_copy(dst_ref.at[pl.ds(base, EB)], dstblk)

            def win(t, _):
                _copy128(cdst128, dstblk, t * GB)
                pltpu.sync_copy(ones128, acc1d.at[cdst128], add=True)
                return 0
            lax.fori_loop(0, EB // GB, win, 0)
            return 0
        lax.fori_loop(0, my_n, blk, 0)
        plsc.subcore_barrier()
        rows = out_rows // NS

        def wb(k, _):
            off = sid * rows + k * 320
            pltpu.sync_copy(acc1d.at[pl.ds(off, 320)], cntbuf)
            pltpu.sync_copy(cntbuf, out.at[pl.ds(off, 320)])
            return 0
        lax.fori_loop(0, rows // 320, wb, 0)
        plsc.subcore_barrier()

    @pl.when(cid == 0)
    def _():
        count_type(dst_ii, E_II // EB, cnt_ii, NIP, NIP)

    @pl.when(cid == 1)
    def _():
        count_type(dst_fi, E_FI // EB, cnt_fi, ACC_B_CNT, NFP)
        count_type(dst_if, E_IF // EB, cnt_if, ACC_B_CNT, NFP)
        count_type(dst_ff, E_FF // EB, cnt_ff, ACC_B_CNT, NFP)


_cnt = functools.partial(
    pl.kernel, _cnt_body, mesh=_mesh, name="sc_counts",
    compiler_params=pltpu.CompilerParams(use_tc_tiling_on_sc=False),
    out_type=(jax.ShapeDtypeStruct((NIP,), jnp.float32),
              jax.ShapeDtypeStruct((NFP,), jnp.float32),
              jax.ShapeDtypeStruct((NFP,), jnp.float32),
              jax.ShapeDtypeStruct((NFP,), jnp.float32)),
    scratch_types=[
        pltpu.VMEM_SHARED((NIP,), jnp.float32),
        pltpu.VMEM((EB,), jnp.int32),
        pltpu.VMEM((GB,), jnp.int32),
        pltpu.VMEM((GB,), jnp.float32),
        pltpu.VMEM((3296,), jnp.float32),
        pltpu.VMEM((320,), jnp.float32),
    ])()


def _bdot(a, b):
    return jnp.dot(a.astype(jnp.bfloat16), b.astype(jnp.bfloat16),
                   preferred_element_type=jnp.float32)


def _proj_kernel(x_ref, w_ref, b_ref, o_ref):
    o_ref[...] = _bdot(x_ref[...], w_ref[...]) + b_ref[...]


def _proj(x, w, b, nrows):
    return pl.pallas_call(
        _proj_kernel,
        grid=(nrows // 1024,),
        in_specs=[pl.BlockSpec((1024, 8), lambda i: (i, 0)),
                  pl.BlockSpec((8, H), lambda i: (0, 0)),
                  pl.BlockSpec((1, H), lambda i: (0, 0))],
        out_specs=pl.BlockSpec((1024, H), lambda i: (i, 0)),
        out_shape=jax.ShapeDtypeStruct((nrows, H), jnp.float32),
    )(x, w, b)


def _dense_kernel(h_ref, agg_a_ref, cnt_a_ref, wl_a_ref,
                  agg_b_ref, cnt_b_ref, wl_b_ref,
                  wr_ref, bias_ref, g_ref, beta_ref, o_ref, *, nb_active):
    b = pl.program_id(0)
    agg_a = jnp.concatenate([agg_a_ref[k] for k in range(4)], axis=-1)
    agg_b = jnp.concatenate([agg_b_ref[k] for k in range(4)], axis=-1)
    mean_a = agg_a / jnp.maximum(cnt_a_ref[...], 1.0)
    x = _bdot(mean_a, wl_a_ref[...])
    x += _bdot(h_ref[...], wr_ref[...])
    mean_b = agg_b / jnp.maximum(cnt_b_ref[...], 1.0)
    mult = jnp.where(b < nb_active, 1.0, 0.0)
    x += mult * _bdot(mean_b, wl_b_ref[...])
    x += bias_ref[...]
    m = jnp.mean(x, axis=-1, keepdims=True)
    v = jnp.mean((x - m) ** 2, axis=-1, keepdims=True)
    y = (x - m) * lax.rsqrt(v + 1e-5) * g_ref[...] + beta_ref[...]
    o_ref[...] = jnp.maximum(h_ref[...] + y, 0.0)


def _dense(h, agg_a, cnt_a, wl_a, agg_b, cnt_b, wl_b, wr, bias, g, beta,
           nb_active):
    nrows = h.shape[0]
    nb_b = agg_b.shape[1] // 1024
    blk = lambda i: (i, 0)
    fix = lambda i: (0, 0)
    return pl.pallas_call(
        functools.partial(_dense_kernel, nb_active=nb_active),
        grid=(nrows // 1024,),
        in_specs=[pl.BlockSpec((1024, H), blk),
                  pl.BlockSpec((4, 1024, CB), lambda i: (0, i, 0)),
                  pl.BlockSpec((1024, 1), blk),
                  pl.BlockSpec((H, H), fix),
                  pl.BlockSpec((4, 1024, CB),
                               lambda i: (0, jnp.minimum(i, nb_b - 1), 0)),
                  pl.BlockSpec((1024, 1),
                               lambda i: (jnp.minimum(i, nb_b - 1), 0)),
                  pl.BlockSpec((H, H), fix),
                  pl.BlockSpec((H, H), fix),
                  pl.BlockSpec((1, H), fix),
                  pl.BlockSpec((1, H), fix),
                  pl.BlockSpec((1, H), fix)],
        out_specs=pl.BlockSpec((1024, H), blk),
        out_shape=jax.ShapeDtypeStruct((nrows, H), jnp.float32),
    )(h, agg_a, cnt_a, wl_a, agg_b, cnt_b, wl_b, wr, bias, g, beta)


def _pool_kernel(h_ref, o_ref):
    b = pl.program_id(0)
    rowid = b * 1024 + lax.broadcasted_iota(jnp.int32, (1024, 1), 0)
    s = jnp.sum(jnp.where(rowid < N_IND, h_ref[...], 0.0),
                axis=0, keepdims=True)

    @pl.when(b == 0)
    def _():
        o_ref[...] = s

    @pl.when(b > 0)
    def _():
        o_ref[...] += s


def _pool(h):
    return pl.pallas_call(
        _pool_kernel,
        grid=(NIP // 1024,),
        in_specs=[pl.BlockSpec((1024, H), lambda i: (i, 0))],
        out_specs=pl.BlockSpec((1, H), lambda i: (0, 0)),
        out_shape=jax.ShapeDtypeStruct((1, H), jnp.float32),
    )(h)


def _heads_kernel(hf_ref, pi_ref, gpp_ref,
                  aw0h_ref, aw0g_ref, ab0_ref, aw1_ref, ab1_ref, aw2_ref,
                  ab2_ref, cw0i_ref, cw0f_ref, cw0g_ref, cb0_ref, cw1_ref,
                  cb1_ref, cw2_ref, cb2_ref,
                  logits_ref, value_ref):
    hf = hf_ref[...]
    gpp = gpp_ref[...]
    dot = _bdot
    a = dot(hf, aw0h_ref[...]) + dot(gpp, aw0g_ref[...]) + ab0_ref[...]
    a = jnp.maximum(a, 0.0)
    a = jnp.maximum(dot(a, aw1_ref[...]) + ab1_ref[...], 0.0)
    logits_ref[...] = dot(a, aw2_ref[...]) + ab2_ref[...]

    rowid = lax.broadcasted_iota(jnp.int32, (NFP, 1), 0)
    pf = jnp.sum(jnp.where(rowid < N_FAC, hf, 0.0), axis=0,
                 keepdims=True) * (1.0 / N_FAC)
    pi = pi_ref[...] * (1.0 / N_IND)
    c = (dot(pi, cw0i_ref[...]) + dot(pf, cw0f_ref[...])
         + dot(gpp, cw0g_ref[...]) + cb0_ref[...])
    c = jnp.maximum(c, 0.0)
    c = jnp.maximum(dot(c, cw1_ref[...]) + cb1_ref[...], 0.0)
    value_ref[...] = dot(c, cw2_ref[...]) + cb2_ref[...]


def _colblocks(h):
    return tuple(h[:, k * CB:(k + 1) * CB] for k in range(4))


def kernel(x_individual, x_facility, edge_index_interacts, edge_index_visits,
           edge_index_visited_by, edge_index_connects, global_features,
           problem_params, params):
    p = params
    f32 = jnp.float32

    xi = jnp.zeros((NIP, 8), f32).at[:N_IND, :5].set(x_individual)
    xf = jnp.zeros((NFP, 8), f32).at[:N_FAC, :3].set(x_facility)
    wi = jnp.zeros((8, H), f32).at[:5].set(p['proj_i_W'])
    wf = jnp.zeros((8, H), f32).at[:3].set(p['proj_f_W'])

    eii = _pad_edges(edge_index_interacts, E_II, N_IND, N_IND, 1024)
    eif = _pad_edges(edge_index_visits, E_IF, N_FAC, NFP, NS)
    efi = _pad_edges(edge_index_visited_by, E_FI, N_FAC, NFP, NS)
    eff = _pad_edges(edge_index_connects, E_FF, N_FAC, NFP, NS)
    s_ii, d_ii = eii[0], eii[1]
    s_fi, d_fi = efi[0], efi[1]
    s_if, d_if = eif[0], eif[1]
    s_ff, d_ff = eff[0], eff[1]

    h_i = _proj(xi, wi, p['proj_i_b'][None, :], NIP)
    h_f = _proj(xf, wf, p['proj_f_b'][None, :], NFP)

    cnt_ii, cnt_fi, cnt_if, cnt_ff = _cnt(d_ii, d_fi, d_if, d_ff)
    cnt_ii = cnt_ii[:, None]
    cnt_fi = cnt_fi[:, None]
    cnt_if = cnt_if[:, None]
    cnt_ff = cnt_ff[:, None]

    for l in range(LAYERS):
        hic = _colblocks(h_i)
        hfc = _colblocks(h_f)
        agg_ii, agg_fi, agg_if, agg_ff = _agg(
            *hic, *hfc, s_ii, d_ii, s_fi, d_fi, s_if, d_if, s_ff, d_ff)
        h_i = _dense(h_i, agg_ii, cnt_ii, p['W_l_%d_ii' % l],
                     agg_fi, cnt_fi, p['W_l_%d_fi' % l],
                     p['W_r_%d_ii' % l] + p['W_r_%d_fi' % l],
                     (p['b_l_%d_ii' % l] + p['b_l_%d_fi' % l])[None, :],
                     p['ln_i_g_%d' % l][None, :], p['ln_i_b_%d' % l][None, :],
                     nb_active=5)
        h_f = _dense(h_f, agg_if, cnt_if, p['W_l_%d_if' % l],
                     agg_ff, cnt_ff, p['W_l_%d_ff' % l],
                     p['W_r_%d_if' % l] + p['W_r_%d_ff' % l],
                     (p['b_l_%d_if' % l] + p['b_l_%d_ff' % l])[None, :],
                     p['ln_f_g_%d' % l][None, :], p['ln_f_b_%d' % l][None, :],
                     nb_active=5)

    pooled_i_sum = _pool(h_i)
    gpp = jnp.zeros((1, 24), f32).at[0, :DG + DP].set(
        jnp.concatenate([global_features, problem_params]))
    aw0g = jnp.zeros((24, H), f32).at[:DG + DP].set(p['actor_W0'][H:])
    aw1 = jnp.zeros((H, H), f32).at[:, :H // 2].set(p['actor_W1'])
    ab1 = jnp.zeros((1, H), f32).at[0, :H // 2].set(p['actor_b1'])
    aw2 = jnp.zeros((H, H), f32).at[:H // 2, 0].set(p['actor_W2'][:, 0])
    ab2 = jnp.zeros((1, H), f32).at[0, 0].set(p['actor_b2'][0])
    cw0g = jnp.zeros((24, H), f32).at[:DG + DP].set(p['critic_W0'][2 * H:])
    cw1 = jnp.zeros((H, H), f32).at[:, :H // 2].set(p['critic_W1'])
    cb1 = jnp.zeros((1, H), f32).at[0, :H // 2].set(p['critic_b1'])
    cw2 = jnp.zeros((H, H), f32).at[:H // 2, 0].set(p['critic_W2'][:, 0])
    cb2 = jnp.zeros((1, H), f32).at[0, 0].set(p['critic_b2'][0])
    logits2, value2 = pl.pallas_call(
        _heads_kernel,
        out_shape=(jax.ShapeDtypeStruct((NFP, H), f32),
                   jax.ShapeDtypeStruct((1, H), f32)),
    )(h_f, pooled_i_sum, gpp,
      p['actor_W0'][:H], aw0g, p['actor_b0'][None, :], aw1, ab1, aw2, ab2,
      p['critic_W0'][:H], p['critic_W0'][H:2 * H], cw0g,
      p['critic_b0'][None, :], cw1, cb1, cw2, cb2)
    return (logits2[:N_FAC, 0], value2[0, 0])
